# Initial kernel scaffold; baseline (speedup 1.0000x reference)
import jax
import jax.numpy as jnp
from jax.experimental import pallas as pl


def kernel(x, gate_weight, gate_bias, keys, key_bias, values, value_bias, s_keys, s_key_bias, s_values, s_value_bias):
    raise NotImplementedError("write your pallas kernel here")


# speedup vs baseline: 2.1818x; 2.1818x over previous
import functools

import jax
import jax.numpy as jnp
from jax import lax
from jax.experimental import pallas as pl
from jax.experimental.pallas import tpu as pltpu
from jax.experimental.pallas import tpu_sc as plsc

D = 1024
H = 4096
E = 8
T = 2048
K = 2
BT = 128
GR = (T * K + E * (BT - 1) + BT - 1) // BT
R = GR * BT
TB = 256
NC, NS = 2, 16
NW = NC * NS
TPW = T // NW
CH = 16
GW = 128


def _router_body(x_ref, gw_ref, gb_ref,
                 g0_ref, g1_ref, e0_ref, e1_ref, r0_ref, r1_ref, cnt_ref,
                 carry):
    i = pl.program_id(0)

    @pl.when(i == 0)
    def _():
        carry[...] = jnp.zeros_like(carry)

    logits = jnp.dot(x_ref[...], gw_ref[...],
                     preferred_element_type=jnp.float32) + gb_ref[...]
    m = jnp.max(logits, axis=1, keepdims=True)
    ex = jnp.exp(logits - m)
    p = ex / jnp.sum(ex, axis=1, keepdims=True)

    ua = lax.broadcasted_iota(jnp.int32, (E, E), 0)
    ub = lax.broadcasted_iota(jnp.int32, (E, E), 1)
    triu = (ua <= ub).astype(jnp.float32)
    lane = lax.broadcasted_iota(jnp.int32, (TB, E), 1).astype(jnp.float32)

    def pick_first_max(q):
        mv = jnp.max(q, axis=1, keepdims=True)
        eq = (q == mv).astype(jnp.float32)
        csum = jnp.dot(eq, triu, preferred_element_type=jnp.float32)
        oh = eq * (csum == 1.0).astype(jnp.float32)
        idx = jnp.sum(oh * lane, axis=1, keepdims=True)
        return mv, oh, idx

    m0, oh0, i0 = pick_first_max(p)
    m1, oh1, i1 = pick_first_max(p - 2.0 * oh0)

    ones16 = jnp.ones((1, GW), jnp.float32)
    g0_ref[...] = m0 * ones16
    g1_ref[...] = m1 * ones16
    e0_ref[...] = i0.astype(jnp.int32)
    e1_ref[...] = i1.astype(jnp.int32)

    ohs = oh0 + oh1
    ra = lax.broadcasted_iota(jnp.int32, (TB, TB), 0)
    rb = lax.broadcasted_iota(jnp.int32, (TB, TB), 1)
    stril = (ra > rb).astype(jnp.float32)
    prior = carry[...] + jnp.dot(stril, ohs, preferred_element_type=jnp.float32)
    r0 = jnp.sum(oh0 * prior, axis=1, keepdims=True)
    r1 = jnp.sum(oh1 * (prior + oh0), axis=1, keepdims=True)
    r0_ref[...] = r0.astype(jnp.int32)
    r1_ref[...] = r1.astype(jnp.int32)

    carry[...] = carry[...] + jnp.sum(ohs, axis=0, keepdims=True)
    cnt_ref[...] = carry[...]


def _router(x2d, gw, gb2d):
    nblk = T // TB
    return pl.pallas_call(
        _router_body,
        grid=(nblk,),
        in_specs=[
            pl.BlockSpec((TB, D), lambda i: (i, 0)),
            pl.BlockSpec((D, E), lambda i: (0, 0)),
            pl.BlockSpec((1, E), lambda i: (0, 0)),
        ],
        out_specs=[
            pl.BlockSpec((TB, GW), lambda i: (i, 0)),
            pl.BlockSpec((TB, GW), lambda i: (i, 0)),
            pl.BlockSpec((TB, 1), lambda i: (i, 0)),
            pl.BlockSpec((TB, 1), lambda i: (i, 0)),
            pl.BlockSpec((TB, 1), lambda i: (i, 0)),
            pl.BlockSpec((TB, 1), lambda i: (i, 0)),
            pl.BlockSpec((1, E), lambda i: (0, 0)),
        ],
        out_shape=[
            jax.ShapeDtypeStruct((T, GW), jnp.float32),
            jax.ShapeDtypeStruct((T, GW), jnp.float32),
            jax.ShapeDtypeStruct((T, 1), jnp.int32),
            jax.ShapeDtypeStruct((T, 1), jnp.int32),
            jax.ShapeDtypeStruct((T, 1), jnp.int32),
            jax.ShapeDtypeStruct((T, 1), jnp.int32),
            jax.ShapeDtypeStruct((1, E), jnp.float32),
        ],
        scratch_shapes=[pltpu.VMEM((1, E), jnp.float32)],
        compiler_params=pltpu.CompilerParams(
            dimension_semantics=("arbitrary",)),
    )(x2d, gw, gb2d)


def _meta_body(cnt_ref, e0_ref, e1_ref, r0_ref, r1_ref,
               pos0_ref, pos1_ref, blk_ref):
    cnt = cnt_ref[...]
    pad = jnp.ceil(cnt / BT) * BT
    ua = lax.broadcasted_iota(jnp.int32, (E, E), 0)
    ub = lax.broadcasted_iota(jnp.int32, (E, E), 1)
    triu = (ua <= ub).astype(jnp.float32)
    upper = jnp.dot(pad, triu, preferred_element_type=jnp.float32)
    off = upper - pad

    lane = lax.broadcasted_iota(jnp.int32, (T, E), 1)

    def to_pos(e_ref, r_ref):
        oh = (lane == e_ref[...]).astype(jnp.float32)
        return (r_ref[...] +
                jnp.sum(oh * off, axis=1, keepdims=True).astype(jnp.int32))

    pos0_ref[...] = to_pos(e0_ref, r0_ref)
    pos1_ref[...] = to_pos(e1_ref, r1_ref)

    brow = lax.broadcasted_iota(jnp.int32, (GR, E), 0).astype(jnp.float32) * BT
    ge = (brow >= upper).astype(jnp.float32)
    blk = jnp.minimum(jnp.sum(ge, axis=1, keepdims=True), float(E - 1))
    blk_ref[...] = blk.astype(jnp.int32)


def _meta(cnt, e0, e1, r0, r1):
    return pl.pallas_call(
        _meta_body,
        out_shape=[
            jax.ShapeDtypeStruct((T, 1), jnp.int32),
            jax.ShapeDtypeStruct((T, 1), jnp.int32),
            jax.ShapeDtypeStruct((GR, 1), jnp.int32),
        ],
    )(cnt, e0, e1, r0, r1)


def _dispatch(x2d, g0s, g1s, pos0w, pos1w):
    mesh = plsc.VectorSubcoreMesh(core_axis_name="c", subcore_axis_name="s",
                                  num_cores=NC, num_subcores=NS)

    @functools.partial(
        pl.kernel,
        out_type=[
            jax.ShapeDtypeStruct((R, D), jnp.float32),
            jax.ShapeDtypeStruct((R, GW), jnp.float32),
        ],
        mesh=mesh,
        scratch_types=[
            pltpu.VMEM((TPW, D), jnp.float32),
            pltpu.VMEM((TPW, GW), jnp.float32),
            pltpu.VMEM((TPW,), jnp.int32),
            pltpu.VMEM((TPW,), jnp.int32),
            pltpu.SemaphoreType.DMA,
        ],
    )
    def k(x_hbm, g0_hbm, g1_hbm, p0_hbm, p1_hbm, xs_hbm, gs_hbm,
          xbuf, gbuf, idx0, idx1, sem):
        wid = lax.axis_index("s") * NC + lax.axis_index("c")
        base = wid * TPW
        pltpu.sync_copy(p0_hbm.at[wid], idx0)
        pltpu.sync_copy(p1_hbm.at[wid], idx1)
        pltpu.sync_copy(x_hbm.at[pl.ds(base, TPW)], xbuf)
        pltpu.async_copy(xbuf, xs_hbm.at[idx0], sem).wait()
        pltpu.async_copy(xbuf, xs_hbm.at[idx1], sem).wait()
        pltpu.sync_copy(g0_hbm.at[pl.ds(base, TPW)], gbuf)
        pltpu.async_copy(gbuf, gs_hbm.at[idx0], sem).wait()
        pltpu.sync_copy(g1_hbm.at[pl.ds(base, TPW)], gbuf)
        pltpu.async_copy(gbuf, gs_hbm.at[idx1], sem).wait()

    return k(x2d, g0s, g1s, pos0w, pos1w)


def _gffn_body(blk_ref, xs_ref, gs_ref, kb_ref, kbias_ref, vb_ref, vbias_ref,
               ys_ref):
    xb = xs_ref[...].astype(jnp.bfloat16)
    h = jnp.dot(xb, kb_ref[...], preferred_element_type=jnp.float32)
    h = jax.nn.gelu(h + kbias_ref[...])
    y = jnp.dot(h.astype(jnp.bfloat16), vb_ref[...],
                preferred_element_type=jnp.float32)
    ys_ref[...] = (y + vbias_ref[...]) * gs_ref[:, 0:1]


def _gffn(blk, xs, gs, kb, kbias, vb, vbias):
    spec = pltpu.PrefetchScalarGridSpec(
        num_scalar_prefetch=1,
        grid=(GR,),
        in_specs=[
            pl.BlockSpec((BT, D), lambda i, b: (i, 0)),
            pl.BlockSpec((BT, GW), lambda i, b: (i, 0)),
            pl.BlockSpec((D, H), lambda i, b: (0, b[i])),
            pl.BlockSpec((1, H), lambda i, b: (0, b[i])),
            pl.BlockSpec((H, D), lambda i, b: (0, b[i])),
            pl.BlockSpec((1, D), lambda i, b: (0, b[i])),
        ],
        out_specs=pl.BlockSpec((BT, D), lambda i, b: (i, 0)),
    )
    return pl.pallas_call(
        _gffn_body,
        grid_spec=spec,
        out_shape=jax.ShapeDtypeStruct((R, D), jnp.float32),
        compiler_params=pltpu.CompilerParams(
            dimension_semantics=("arbitrary",)),
    )(blk, xs, gs, kb, kbias, vb, vbias)


def _sffn_body(x_ref, sk_ref, skb_ref, sv_ref, svb_ref, o_ref):
    xb = x_ref[...].astype(jnp.bfloat16)
    h = jnp.dot(xb, sk_ref[...], preferred_element_type=jnp.float32)
    h = jax.nn.gelu(h + skb_ref[...])
    y = jnp.dot(h.astype(jnp.bfloat16), sv_ref[...],
                preferred_element_type=jnp.float32)
    o_ref[...] = y + svb_ref[...]


def _sffn(x2d, sk, skb, sv, svb):
    nblk = T // BT
    return pl.pallas_call(
        _sffn_body,
        grid=(nblk,),
        in_specs=[
            pl.BlockSpec((BT, D), lambda i: (i, 0)),
            pl.BlockSpec((D, H), lambda i: (0, 0)),
            pl.BlockSpec((1, H), lambda i: (0, 0)),
            pl.BlockSpec((H, D), lambda i: (0, 0)),
            pl.BlockSpec((1, D), lambda i: (0, 0)),
        ],
        out_specs=pl.BlockSpec((BT, D), lambda i: (i, 0)),
        out_shape=jax.ShapeDtypeStruct((T, D), jnp.float32),
    )(x2d, sk, skb, sv, svb)


def _combine(ys, sh, pos0w, pos1w):
    mesh = plsc.VectorSubcoreMesh(core_axis_name="c", subcore_axis_name="s",
                                  num_cores=NC, num_subcores=NS)

    @functools.partial(
        pl.kernel,
        out_type=jax.ShapeDtypeStruct((T, D), jnp.float32),
        mesh=mesh,
        scratch_types=[
            pltpu.VMEM((CH, D), jnp.float32),
            pltpu.VMEM((CH, D), jnp.float32),
            pltpu.VMEM((CH, D), jnp.float32),
            pltpu.VMEM((TPW,), jnp.int32),
            pltpu.VMEM((TPW,), jnp.int32),
            pltpu.SemaphoreType.DMA,
        ],
    )
    def k(ys_hbm, sh_hbm, p0_hbm, p1_hbm, out_hbm,
          shbuf, b0, b1, idx0, idx1, sem):
        wid = lax.axis_index("s") * NC + lax.axis_index("c")
        base = wid * TPW
        pltpu.sync_copy(p0_hbm.at[wid], idx0)
        pltpu.sync_copy(p1_hbm.at[wid], idx1)
        for j in range(TPW // CH):
            off = j * CH
            iv0 = idx0[pl.ds(off, CH)]
            iv1 = idx1[pl.ds(off, CH)]
            pltpu.sync_copy(sh_hbm.at[pl.ds(base + off, CH)], shbuf)
            pltpu.async_copy(ys_hbm.at[iv0], b0, sem).wait()
            pltpu.async_copy(ys_hbm.at[iv1], b1, sem).wait()

            def row_body(r, _):
                def col_body(c, _):
                    sl = pl.ds(c * 16, 16)
                    shbuf[r, sl] = shbuf[r, sl] + b0[r, sl] + b1[r, sl]
                    return 0
                lax.fori_loop(0, D // 16, col_body, 0)
                return 0

            lax.fori_loop(0, CH, row_body, 0)
            pltpu.sync_copy(shbuf, out_hbm.at[pl.ds(base + off, CH)])

    return k(ys, sh, pos0w, pos1w)


def kernel(x, gate_weight, gate_bias, keys, key_bias, values, value_bias,
           s_keys, s_key_bias, s_values, s_value_bias):
    x2d = x.reshape(T, D)
    gb2d = gate_bias.reshape(1, E)
    kb = keys.astype(jnp.bfloat16).reshape(D, E * H)
    vb = values.astype(jnp.bfloat16).reshape(H, E * D)
    sk = s_keys.astype(jnp.bfloat16).reshape(D, H)
    sv = s_values.astype(jnp.bfloat16).reshape(H, D)
    kbias = key_bias.reshape(1, E * H)
    vbias = value_bias.reshape(1, E * D)
    skb = s_key_bias.reshape(1, H)
    svb = s_value_bias.reshape(1, D)

    g0s, g1s, e0, e1, r0, r1, cnt = _router(x2d, gate_weight, gb2d)
    pos0, pos1, blk = _meta(cnt, e0, e1, r0, r1)
    pos0w = pos0.reshape(NW, TPW)
    pos1w = pos1.reshape(NW, TPW)

    xs, gs = _dispatch(x2d, g0s, g1s, pos0w, pos1w)
    ys = _gffn(blk.reshape(GR), xs, gs, kb, kbias, vb, vbias)
    sh = _sffn(x2d, sk, skb, sv, svb)
    out = _combine(ys, sh, pos0w, pos1w)
    return out.reshape(x.shape)
